# Initial kernel scaffold; baseline (speedup 1.0000x reference)
import jax
import jax.numpy as jnp
from jax.experimental import pallas as pl


def kernel(x, edge_index, W, b):
    raise NotImplementedError("write your pallas kernel here")


# speedup vs baseline: 23.2519x; 23.2519x over previous
import functools

import jax
import jax.numpy as jnp
from jax import lax
from jax.experimental import pallas as pl
from jax.experimental.pallas import tpu as pltpu
from jax.experimental.pallas import tpu_sc as plsc

N = 10000
E = 320000
D = 128
NC = 2
NS = 16
NW = NC * NS
EPW = E // NW
BATCH = 80
NB = EPW // BATCH
RQ = 624
TAIL_BASE = RQ * NS
TAIL = N - TAIL_BASE
DEGW = 128


def _striped_copy(src, dst, s):
    base = s * RQ
    pltpu.sync_copy(src.at[pl.ds(base, RQ)], dst.at[pl.ds(base, RQ)])

    @pl.when(s == NS - 1)
    def _():
        pltpu.sync_copy(src.at[pl.ds(TAIL_BASE, TAIL)],
                        dst.at[pl.ds(TAIL_BASE, TAIL)])

_mesh = plsc.VectorSubcoreMesh(core_axis_name="c", subcore_axis_name="s")


def _deg_body(dst_hbm, ones_hbm, zeros_hbm, deg_hbm, shared_deg, idx_v, ones_v):
    c = lax.axis_index("c")
    s = lax.axis_index("s")
    wid = s * NC + c
    _striped_copy(zeros_hbm, shared_deg, s)
    pltpu.sync_copy(ones_hbm, ones_v)
    pltpu.sync_copy(dst_hbm.at[wid], idx_v)
    plsc.subcore_barrier()

    def body(i, carry):
        pltpu.sync_copy(ones_v, shared_deg.at[idx_v.at[i]], add=True)
        return carry

    lax.fori_loop(0, NB, body, 0)
    plsc.subcore_barrier()
    _striped_copy(shared_deg, deg_hbm.at[c], s)


_deg_kernel = functools.partial(
    pl.kernel,
    out_type=jax.ShapeDtypeStruct((NC, N, DEGW), jnp.float32),
    mesh=_mesh,
    scratch_types=[
        pltpu.VMEM_SHARED((N, DEGW), jnp.float32),
        pltpu.VMEM((NB, BATCH), jnp.int32),
        pltpu.VMEM((BATCH, DEGW), jnp.float32),
    ],
)(_deg_body)


def _agg_body(src_hbm, dst_hbm, y_hbm, zeros_hbm, agg_hbm,
              shared_agg, sidx_v, didx_v, rows_v, sem):
    c = lax.axis_index("c")
    s = lax.axis_index("s")
    wid = s * NC + c

    @pl.when(c == 0)
    def _():
        _striped_copy(y_hbm, shared_agg, s)

    @pl.when(c != 0)
    def _():
        _striped_copy(zeros_hbm, shared_agg, s)

    pltpu.sync_copy(src_hbm.at[wid], sidx_v)
    pltpu.sync_copy(dst_hbm.at[wid], didx_v)
    plsc.subcore_barrier()

    def body(i, carry):
        pltpu.async_copy(y_hbm.at[sidx_v.at[i]], rows_v, sem).wait()
        pltpu.sync_copy(rows_v, shared_agg.at[didx_v.at[i]], add=True)
        return carry

    lax.fori_loop(0, NB, body, 0)
    plsc.subcore_barrier()
    _striped_copy(shared_agg, agg_hbm.at[c], s)


_agg_kernel = functools.partial(
    pl.kernel,
    out_type=jax.ShapeDtypeStruct((NC, N, D), jnp.float32),
    mesh=_mesh,
    scratch_types=[
        pltpu.VMEM_SHARED((N, D), jnp.float32),
        pltpu.VMEM((NB, BATCH), jnp.int32),
        pltpu.VMEM((NB, BATCH), jnp.int32),
        pltpu.VMEM((BATCH, D), jnp.float32),
        pltpu.SemaphoreType.DMA,
    ],
)(_agg_body)


def _mm_body(x_ref, w_ref, deg_ref, y_ref):
    deg = deg_ref[0, :, 0:1] + deg_ref[1, :, 0:1] + 1.0
    dis = lax.rsqrt(deg)
    xw = jnp.dot(x_ref[...], w_ref[...], preferred_element_type=jnp.float32)
    y_ref[...] = xw * dis


def _mm_kernel(x, w, deg):
    return pl.pallas_call(
        _mm_body,
        out_shape=jax.ShapeDtypeStruct((N, D), jnp.float32),
    )(x, w, deg)


def _fin_body(agg_ref, deg_ref, b_ref, out_ref):
    dis = lax.rsqrt(deg_ref[0, :, 0:1] + deg_ref[1, :, 0:1] + 1.0)
    out_ref[...] = (agg_ref[0] + agg_ref[1]) * dis + b_ref[...]


def _fin_kernel(agg, deg, b):
    return pl.pallas_call(
        _fin_body,
        out_shape=jax.ShapeDtypeStruct((N, D), jnp.float32),
    )(agg, deg, b)


def kernel(x, edge_index, W, b):
    ei = edge_index.astype(jnp.int32)
    src = ei[0].reshape(NW, NB, BATCH)
    dst = ei[1].reshape(NW, NB, BATCH)
    ones16 = jnp.ones((BATCH, DEGW), jnp.float32)
    zeros16 = jnp.zeros((N, DEGW), jnp.float32)
    zeros128 = jnp.zeros((N, D), jnp.float32)

    deg2 = _deg_kernel(dst, ones16, zeros16)
    y = _mm_kernel(x, W, deg2)
    agg2 = _agg_kernel(src, dst, y, zeros128)
    return _fin_kernel(agg2, deg2, b.reshape(1, D))
